# baseline (speedup 1.0000x reference)
import functools

import jax
import jax.numpy as jnp
from jax import lax
from jax.experimental import pallas as pl
from jax.experimental.pallas import tpu as pltpu
from jax.experimental.pallas import tpu_sc as plsc

_B, _S, _DIM = 2, 2048, 1024
_E, _H, _TOPK = 8, 4096, 2
_N = _B * _S
_Z_LOSS_COEF = 0.001

_TOK_BLK = 512
_H_BLK = 2048
_M_TILE = 128
_N_TILES = (_TOPK * _N) // _M_TILE + _E
_NPAD = _N_TILES * _M_TILE

_SC_NC, _SC_NS = 2, 16
_SC_NW = _SC_NC * _SC_NS
_T_W = _N // _SC_NW
_CH = 64


def _router_kernel(x_ref, wr_ref, a1_ref, a2_ref, p1_ref, p2_ref,
                   cnt_ref, zsum_ref, psum_ref):
    t = pl.program_id(0)
    x = x_ref[...]
    logits = lax.dot_general(
        x, wr_ref[...], (((1,), (1,)), ((), ())),
        preferred_element_type=jnp.float32)

    @pl.when(t == 0)
    def _init():
        cnt_ref[...] = jnp.zeros_like(cnt_ref)
        zsum_ref[...] = jnp.zeros_like(zsum_ref)
        psum_ref[...] = jnp.zeros_like(psum_ref)

    zsum_ref[...] += jnp.sum(logits * logits).reshape(1, 1)

    m1 = jnp.max(logits, axis=-1, keepdims=True)
    ex = jnp.exp(logits - m1)
    probs = ex / jnp.sum(ex, axis=-1, keepdims=True)
    psum_ref[...] += jnp.sum(probs, axis=0, keepdims=True)

    idx = lax.broadcasted_iota(jnp.int32, logits.shape, 1)
    big = jnp.int32(2**30)
    a1 = jnp.min(jnp.where(logits == m1, idx, big), axis=-1, keepdims=True)
    l2 = jnp.where(idx == a1, -jnp.inf, logits)
    m2 = jnp.max(l2, axis=-1, keepdims=True)
    a2 = jnp.min(jnp.where(l2 == m2, idx, big), axis=-1, keepdims=True)
    p1 = jax.nn.sigmoid(m1 - m2)

    a1_ref[...] = a1
    a2_ref[...] = a2
    p1_ref[...] = p1
    p2_ref[...] = 1.0 - p1

    onehot = (jnp.where(idx == a1, 1.0, 0.0)
              + jnp.where(idx == a2, 1.0, 0.0))
    cnt_ref[...] += jnp.sum(onehot, axis=0, keepdims=True).astype(jnp.int32)


def _dispatch_kernel(a1_ref, a2_ref, base_ref, d1_ref, d2_ref, run_ref):
    t = pl.program_id(0)

    @pl.when(t == 0)
    def _init():
        run_ref[...] = jnp.zeros_like(run_ref)

    a1 = a1_ref[...]
    a2 = a2_ref[...]
    lane = lax.broadcasted_iota(jnp.int32, (_TOK_BLK, _E), 1)
    onehot = (jnp.where(lane == a1, 1.0, 0.0)
              + jnp.where(lane == a2, 1.0, 0.0))
    r_io = lax.broadcasted_iota(jnp.int32, (_TOK_BLK, _TOK_BLK), 0)
    c_io = lax.broadcasted_iota(jnp.int32, (_TOK_BLK, _TOK_BLK), 1)
    tril = jnp.where(c_io < r_io, 1.0, 0.0)
    excl = lax.dot_general(
        tril, onehot, (((1,), (0,)), ((), ())),
        preferred_element_type=jnp.float32)
    pos = excl + run_ref[...] + base_ref[...].astype(jnp.float32)
    d1 = jnp.sum(jnp.where(lane == a1, pos, 0.0), axis=1, keepdims=True)
    d2 = jnp.sum(jnp.where(lane == a2, pos, 0.0), axis=1, keepdims=True)
    d1_ref[...] = d1.astype(jnp.int32)
    d2_ref[...] = d2.astype(jnp.int32)
    run_ref[...] += jnp.sum(onehot, axis=0, keepdims=True)


def _sc_scatter_call(xf, d1, d2):
    mesh = plsc.VectorSubcoreMesh(core_axis_name="c", subcore_axis_name="s")

    @functools.partial(
        pl.kernel,
        out_type=jax.ShapeDtypeStruct((_NPAD, _DIM), jnp.float32),
        mesh=mesh,
        scratch_types=[
            pltpu.VMEM((_CH,), jnp.int32),
            pltpu.VMEM((_CH,), jnp.int32),
            pltpu.VMEM((_CH, _DIM), jnp.float32),
        ],
    )
    def scatter_k(x_hbm, d1_hbm, d2_hbm, xs_hbm, idx1_v, idx2_v, rows_v):
        wid = lax.axis_index("s") * _SC_NC + lax.axis_index("c")
        base = wid * _T_W
        for c in range(_T_W // _CH):
            off = base + c * _CH
            pltpu.sync_copy(d1_hbm.at[pl.ds(off, _CH)], idx1_v)
            pltpu.sync_copy(d2_hbm.at[pl.ds(off, _CH)], idx2_v)
            pltpu.sync_copy(x_hbm.at[pl.ds(off, _CH)], rows_v)
            pltpu.sync_copy(rows_v, xs_hbm.at[idx1_v])
            pltpu.sync_copy(rows_v, xs_hbm.at[idx2_v])

    return scatter_k(xf, d1, d2)


def _sc_gather_call(os_, d1, d2):
    mesh = plsc.VectorSubcoreMesh(core_axis_name="c", subcore_axis_name="s")

    @functools.partial(
        pl.kernel,
        out_type=[
            jax.ShapeDtypeStruct((_N, _DIM), jnp.float32),
            jax.ShapeDtypeStruct((_N, _DIM), jnp.float32),
        ],
        mesh=mesh,
        scratch_types=[
            pltpu.VMEM((_CH,), jnp.int32),
            pltpu.VMEM((_CH, _DIM), jnp.float32),
            pltpu.SemaphoreType.DMA,
        ],
    )
    def gather_k(os_hbm, d1_hbm, d2_hbm, g1_hbm, g2_hbm, idx_v, rows_v, sem):
        wid = lax.axis_index("s") * _SC_NC + lax.axis_index("c")
        base = wid * _T_W
        for c in range(_T_W // _CH):
            off = base + c * _CH
            pltpu.sync_copy(d1_hbm.at[pl.ds(off, _CH)], idx_v)
            pltpu.async_copy(os_hbm.at[idx_v], rows_v, sem).wait()
            pltpu.sync_copy(rows_v, g1_hbm.at[pl.ds(off, _CH)])
            pltpu.sync_copy(d2_hbm.at[pl.ds(off, _CH)], idx_v)
            pltpu.async_copy(os_hbm.at[idx_v], rows_v, sem).wait()
            pltpu.sync_copy(rows_v, g2_hbm.at[pl.ds(off, _CH)])

    return gather_k(os_, d1, d2)


def _ffn_part(xs_ref, w1_ref, w3_ref, w2_ref):
    x = xs_ref[...]
    hidden = lax.dot_general(
        x, w1_ref[0], (((1,), (1,)), ((), ())),
        preferred_element_type=jnp.float32)
    gate = lax.dot_general(
        x, w3_ref[0], (((1,), (1,)), ((), ())),
        preferred_element_type=jnp.float32)
    hg = hidden * (gate * jax.nn.sigmoid(gate))
    return lax.dot_general(
        hg, w2_ref[0], (((1,), (1,)), ((), ())),
        preferred_element_type=jnp.float32)


def _ffn_first_kernel(te_ref, xs_ref, w1_ref, w3_ref, w2_ref, os_ref):
    @pl.when(pl.program_id(0) < te_ref[_N_TILES])
    def _compute():
        os_ref[...] = _ffn_part(xs_ref, w1_ref, w3_ref, w2_ref)


def _ffn_acc_kernel(te_ref, xs_ref, w1_ref, w3_ref, w2_ref, oi_ref, os_ref):
    @pl.when(pl.program_id(0) < te_ref[_N_TILES])
    def _compute():
        os_ref[...] = oi_ref[...] + _ffn_part(xs_ref, w1_ref, w3_ref, w2_ref)


def _combine_kernel(g1_ref, g2_ref, p1_ref, p2_ref, y_ref):
    y_ref[...] = p1_ref[...] * g1_ref[...] + p2_ref[...] * g2_ref[...]


def kernel(x, Wr, W1, W2, W3):
    xf = x.reshape(_N, _DIM)
    n_tok = _N // _TOK_BLK

    a1, a2, p1, p2, cnt, zsum, psum = pl.pallas_call(
        _router_kernel,
        grid=(n_tok,),
        in_specs=[
            pl.BlockSpec((_TOK_BLK, _DIM), lambda t: (t, 0)),
            pl.BlockSpec((_E, _DIM), lambda t: (0, 0)),
        ],
        out_specs=[
            pl.BlockSpec((_TOK_BLK, 1), lambda t: (t, 0)),
            pl.BlockSpec((_TOK_BLK, 1), lambda t: (t, 0)),
            pl.BlockSpec((_TOK_BLK, 1), lambda t: (t, 0)),
            pl.BlockSpec((_TOK_BLK, 1), lambda t: (t, 0)),
            pl.BlockSpec((1, _E), lambda t: (0, 0)),
            pl.BlockSpec((1, 1), lambda t: (0, 0)),
            pl.BlockSpec((1, _E), lambda t: (0, 0)),
        ],
        out_shape=[
            jax.ShapeDtypeStruct((_N, 1), jnp.int32),
            jax.ShapeDtypeStruct((_N, 1), jnp.int32),
            jax.ShapeDtypeStruct((_N, 1), jnp.float32),
            jax.ShapeDtypeStruct((_N, 1), jnp.float32),
            jax.ShapeDtypeStruct((1, _E), jnp.int32),
            jax.ShapeDtypeStruct((1, 1), jnp.float32),
            jax.ShapeDtypeStruct((1, _E), jnp.float32),
        ],
    )(xf, Wr)

    c = cnt[0]
    tiles = (c + _M_TILE - 1) // _M_TILE
    cum_tiles = jnp.cumsum(tiles)
    slot_base = ((cum_tiles - tiles) * _M_TILE).astype(jnp.int32)[None, :]
    tile_ids = jnp.arange(_N_TILES, dtype=jnp.int32)
    tile_expert = jnp.minimum(
        jnp.sum(tile_ids[:, None] >= cum_tiles[None, :].astype(jnp.int32),
                axis=1),
        _E - 1).astype(jnp.int32)
    tile_expert = jnp.concatenate(
        [tile_expert, cum_tiles[-1:].astype(jnp.int32)])

    d1, d2 = pl.pallas_call(
        _dispatch_kernel,
        grid=(n_tok,),
        in_specs=[
            pl.BlockSpec((_TOK_BLK, 1), lambda t: (t, 0)),
            pl.BlockSpec((_TOK_BLK, 1), lambda t: (t, 0)),
            pl.BlockSpec((1, _E), lambda t: (0, 0)),
        ],
        out_specs=[
            pl.BlockSpec((_TOK_BLK, 1), lambda t: (t, 0)),
            pl.BlockSpec((_TOK_BLK, 1), lambda t: (t, 0)),
        ],
        out_shape=[
            jax.ShapeDtypeStruct((_N, 1), jnp.int32),
            jax.ShapeDtypeStruct((_N, 1), jnp.int32),
        ],
        scratch_shapes=[pltpu.VMEM((1, _E), jnp.float32)],
    )(a1, a2, slot_base)

    d1f = d1.reshape(_N)
    d2f = d2.reshape(_N)

    xs = _sc_scatter_call(xf, d1f, d2f)

    n_h = _H // _H_BLK
    os_ = None
    for h in range(n_h):
        w_specs = [
            pl.BlockSpec((1, _H_BLK, _DIM), lambda t, te, h=h: (te[t], h, 0)),
            pl.BlockSpec((1, _H_BLK, _DIM), lambda t, te, h=h: (te[t], h, 0)),
            pl.BlockSpec((1, _DIM, _H_BLK), lambda t, te, h=h: (te[t], 0, h)),
        ]
        xs_spec = pl.BlockSpec((_M_TILE, _DIM), lambda t, te: (t, 0))
        o_spec = pl.BlockSpec((_M_TILE, _DIM), lambda t, te: (t, 0))
        if h == 0:
            os_ = pl.pallas_call(
                _ffn_first_kernel,
                grid_spec=pltpu.PrefetchScalarGridSpec(
                    num_scalar_prefetch=1,
                    grid=(_N_TILES,),
                    in_specs=[xs_spec] + w_specs,
                    out_specs=o_spec,
                ),
                out_shape=jax.ShapeDtypeStruct((_NPAD, _DIM), jnp.float32),
            )(tile_expert, xs, W1, W3, W2)
        else:
            os_ = pl.pallas_call(
                _ffn_acc_kernel,
                grid_spec=pltpu.PrefetchScalarGridSpec(
                    num_scalar_prefetch=1,
                    grid=(_N_TILES,),
                    in_specs=[xs_spec] + w_specs + [o_spec],
                    out_specs=o_spec,
                ),
                out_shape=jax.ShapeDtypeStruct((_NPAD, _DIM), jnp.float32),
                input_output_aliases={5: 0},
            )(tile_expert, xs, W1, W3, W2, os_)

    g1, g2 = _sc_gather_call(os_, d1f, d2f)

    y = pl.pallas_call(
        _combine_kernel,
        grid=(n_tok,),
        in_specs=[
            pl.BlockSpec((_TOK_BLK, _DIM), lambda t: (t, 0)),
            pl.BlockSpec((_TOK_BLK, _DIM), lambda t: (t, 0)),
            pl.BlockSpec((_TOK_BLK, 1), lambda t: (t, 0)),
            pl.BlockSpec((_TOK_BLK, 1), lambda t: (t, 0)),
        ],
        out_specs=pl.BlockSpec((_TOK_BLK, _DIM), lambda t: (t, 0)),
        out_shape=jax.ShapeDtypeStruct((_N, _DIM), jnp.float32),
    )(g1, g2, p1, p2)

    z_loss = (zsum[0, 0] / (_N * _E)) * _Z_LOSS_COEF
    balance_loss = jnp.mean(jnp.square(psum[0] / _N - 1.0 / _E))
    return (y.reshape(_B, _S, _DIM), z_loss, balance_loss)


# speedup vs baseline: 1.6892x; 1.6892x over previous
import functools

import jax
import jax.numpy as jnp
from jax import lax
from jax.experimental import pallas as pl
from jax.experimental.pallas import tpu as pltpu
from jax.experimental.pallas import tpu_sc as plsc

_B, _S, _DIM = 2, 2048, 1024
_E, _H, _TOPK = 8, 4096, 2
_N = _B * _S
_Z_LOSS_COEF = 0.001

_TOK_BLK = 512
_H_BLK = 2048
_M_TILE = 256
_N_TILES = (_TOPK * _N) // _M_TILE + _E
_NPAD = _N_TILES * _M_TILE

_SC_NC, _SC_NS = 2, 16
_SC_NW = _SC_NC * _SC_NS
_T_W = _N // _SC_NW
_CH = 64


def _router_kernel(x_ref, wr_ref, a1_ref, a2_ref, p1_ref, p2_ref,
                   cnt_ref, zsum_ref, psum_ref):
    t = pl.program_id(0)
    x = x_ref[...]
    logits = lax.dot_general(
        x, wr_ref[...], (((1,), (1,)), ((), ())),
        preferred_element_type=jnp.float32)

    @pl.when(t == 0)
    def _init():
        cnt_ref[...] = jnp.zeros_like(cnt_ref)
        zsum_ref[...] = jnp.zeros_like(zsum_ref)
        psum_ref[...] = jnp.zeros_like(psum_ref)

    zsum_ref[...] += jnp.sum(logits * logits).reshape(1, 1)

    m1 = jnp.max(logits, axis=-1, keepdims=True)
    ex = jnp.exp(logits - m1)
    probs = ex / jnp.sum(ex, axis=-1, keepdims=True)
    psum_ref[...] += jnp.sum(probs, axis=0, keepdims=True)

    idx = lax.broadcasted_iota(jnp.int32, logits.shape, 1)
    big = jnp.int32(2**30)
    a1 = jnp.min(jnp.where(logits == m1, idx, big), axis=-1, keepdims=True)
    l2 = jnp.where(idx == a1, -jnp.inf, logits)
    m2 = jnp.max(l2, axis=-1, keepdims=True)
    a2 = jnp.min(jnp.where(l2 == m2, idx, big), axis=-1, keepdims=True)
    p1 = jax.nn.sigmoid(m1 - m2)

    a1_ref[...] = a1
    a2_ref[...] = a2
    p1_ref[...] = p1
    p2_ref[...] = 1.0 - p1

    onehot = (jnp.where(idx == a1, 1.0, 0.0)
              + jnp.where(idx == a2, 1.0, 0.0))
    cnt_ref[...] += jnp.sum(onehot, axis=0, keepdims=True).astype(jnp.int32)


def _dispatch_kernel(a1_ref, a2_ref, base_ref, d1_ref, d2_ref, run_ref):
    t = pl.program_id(0)

    @pl.when(t == 0)
    def _init():
        run_ref[...] = jnp.zeros_like(run_ref)

    a1 = a1_ref[...]
    a2 = a2_ref[...]
    lane = lax.broadcasted_iota(jnp.int32, (_TOK_BLK, _E), 1)
    onehot = (jnp.where(lane == a1, 1.0, 0.0)
              + jnp.where(lane == a2, 1.0, 0.0))
    r_io = lax.broadcasted_iota(jnp.int32, (_TOK_BLK, _TOK_BLK), 0)
    c_io = lax.broadcasted_iota(jnp.int32, (_TOK_BLK, _TOK_BLK), 1)
    tril = jnp.where(c_io < r_io, 1.0, 0.0)
    excl = lax.dot_general(
        tril, onehot, (((1,), (0,)), ((), ())),
        preferred_element_type=jnp.float32)
    pos = excl + run_ref[...] + base_ref[...].astype(jnp.float32)
    d1 = jnp.sum(jnp.where(lane == a1, pos, 0.0), axis=1, keepdims=True)
    d2 = jnp.sum(jnp.where(lane == a2, pos, 0.0), axis=1, keepdims=True)
    d1_ref[...] = d1.astype(jnp.int32)
    d2_ref[...] = d2.astype(jnp.int32)
    run_ref[...] += jnp.sum(onehot, axis=0, keepdims=True)


def _sc_scatter_call(xf, d1, d2):
    mesh = plsc.VectorSubcoreMesh(core_axis_name="c", subcore_axis_name="s")

    @functools.partial(
        pl.kernel,
        out_type=jax.ShapeDtypeStruct((_NPAD, _DIM), jnp.float32),
        mesh=mesh,
        scratch_types=[
            pltpu.VMEM((_CH,), jnp.int32),
            pltpu.VMEM((_CH,), jnp.int32),
            pltpu.VMEM((_CH, _DIM), jnp.float32),
        ],
    )
    def scatter_k(x_hbm, d1_hbm, d2_hbm, xs_hbm, idx1_v, idx2_v, rows_v):
        wid = lax.axis_index("s") * _SC_NC + lax.axis_index("c")
        base = wid * _T_W
        for c in range(_T_W // _CH):
            off = base + c * _CH
            pltpu.sync_copy(d1_hbm.at[pl.ds(off, _CH)], idx1_v)
            pltpu.sync_copy(d2_hbm.at[pl.ds(off, _CH)], idx2_v)
            pltpu.sync_copy(x_hbm.at[pl.ds(off, _CH)], rows_v)
            pltpu.sync_copy(rows_v, xs_hbm.at[idx1_v])
            pltpu.sync_copy(rows_v, xs_hbm.at[idx2_v])

    return scatter_k(xf, d1, d2)


def _sc_gather_call(os_, d1, d2):
    mesh = plsc.VectorSubcoreMesh(core_axis_name="c", subcore_axis_name="s")

    @functools.partial(
        pl.kernel,
        out_type=[
            jax.ShapeDtypeStruct((_N, _DIM), jnp.float32),
            jax.ShapeDtypeStruct((_N, _DIM), jnp.float32),
        ],
        mesh=mesh,
        scratch_types=[
            pltpu.VMEM((_CH,), jnp.int32),
            pltpu.VMEM((_CH, _DIM), jnp.float32),
            pltpu.SemaphoreType.DMA,
        ],
    )
    def gather_k(os_hbm, d1_hbm, d2_hbm, g1_hbm, g2_hbm, idx_v, rows_v, sem):
        wid = lax.axis_index("s") * _SC_NC + lax.axis_index("c")
        base = wid * _T_W
        for c in range(_T_W // _CH):
            off = base + c * _CH
            pltpu.sync_copy(d1_hbm.at[pl.ds(off, _CH)], idx_v)
            pltpu.async_copy(os_hbm.at[idx_v], rows_v, sem).wait()
            pltpu.sync_copy(rows_v, g1_hbm.at[pl.ds(off, _CH)])
            pltpu.sync_copy(d2_hbm.at[pl.ds(off, _CH)], idx_v)
            pltpu.async_copy(os_hbm.at[idx_v], rows_v, sem).wait()
            pltpu.sync_copy(rows_v, g2_hbm.at[pl.ds(off, _CH)])

    return gather_k(os_, d1, d2)


def _ffn_part(xs_ref, w1_ref, w3_ref, w2_ref):
    x = xs_ref[...]
    hidden = lax.dot_general(
        x, w1_ref[0], (((1,), (1,)), ((), ())),
        preferred_element_type=jnp.float32)
    gate = lax.dot_general(
        x, w3_ref[0], (((1,), (1,)), ((), ())),
        preferred_element_type=jnp.float32)
    hg = hidden * (gate * jax.nn.sigmoid(gate))
    return lax.dot_general(
        hg, w2_ref[0], (((1,), (1,)), ((), ())),
        preferred_element_type=jnp.float32)


def _ffn_first_kernel(te_ref, xs_ref, w1_ref, w3_ref, w2_ref, os_ref):
    @pl.when(pl.program_id(0) < te_ref[_N_TILES])
    def _compute():
        os_ref[...] = _ffn_part(xs_ref, w1_ref, w3_ref, w2_ref)


def _ffn_acc_kernel(te_ref, xs_ref, w1_ref, w3_ref, w2_ref, oi_ref, os_ref):
    @pl.when(pl.program_id(0) < te_ref[_N_TILES])
    def _compute():
        os_ref[...] = oi_ref[...] + _ffn_part(xs_ref, w1_ref, w3_ref, w2_ref)


def _combine_kernel(g1_ref, g2_ref, p1_ref, p2_ref, zsum_ref, psum_ref,
                    y_ref, loss_ref):
    y_ref[...] = p1_ref[...] * g1_ref[...] + p2_ref[...] * g2_ref[...]

    @pl.when(pl.program_id(0) == 0)
    def _losses():
        z = (zsum_ref[0, 0] / (_N * _E)) * _Z_LOSS_COEF
        pm = psum_ref[...] / _N - 1.0 / _E
        bal = jnp.sum(pm * pm) / _E
        loss_ref[...] = jnp.concatenate(
            [jnp.full((1, 1), z), jnp.full((1, 1), bal)], axis=1)


def kernel(x, Wr, W1, W2, W3):
    xf = x.reshape(_N, _DIM)
    n_tok = _N // _TOK_BLK

    a1, a2, p1, p2, cnt, zsum, psum = pl.pallas_call(
        _router_kernel,
        grid=(n_tok,),
        in_specs=[
            pl.BlockSpec((_TOK_BLK, _DIM), lambda t: (t, 0)),
            pl.BlockSpec((_E, _DIM), lambda t: (0, 0)),
        ],
        out_specs=[
            pl.BlockSpec((_TOK_BLK, 1), lambda t: (t, 0)),
            pl.BlockSpec((_TOK_BLK, 1), lambda t: (t, 0)),
            pl.BlockSpec((_TOK_BLK, 1), lambda t: (t, 0)),
            pl.BlockSpec((_TOK_BLK, 1), lambda t: (t, 0)),
            pl.BlockSpec((1, _E), lambda t: (0, 0)),
            pl.BlockSpec((1, 1), lambda t: (0, 0)),
            pl.BlockSpec((1, _E), lambda t: (0, 0)),
        ],
        out_shape=[
            jax.ShapeDtypeStruct((_N, 1), jnp.int32),
            jax.ShapeDtypeStruct((_N, 1), jnp.int32),
            jax.ShapeDtypeStruct((_N, 1), jnp.float32),
            jax.ShapeDtypeStruct((_N, 1), jnp.float32),
            jax.ShapeDtypeStruct((1, _E), jnp.int32),
            jax.ShapeDtypeStruct((1, 1), jnp.float32),
            jax.ShapeDtypeStruct((1, _E), jnp.float32),
        ],
    )(xf, Wr)

    c = cnt[0]
    tiles = (c + _M_TILE - 1) // _M_TILE
    cum_tiles = jnp.cumsum(tiles)
    slot_base = ((cum_tiles - tiles) * _M_TILE).astype(jnp.int32)[None, :]
    tile_ids = jnp.arange(_N_TILES, dtype=jnp.int32)
    tile_expert = jnp.minimum(
        jnp.sum(tile_ids[:, None] >= cum_tiles[None, :].astype(jnp.int32),
                axis=1),
        _E - 1).astype(jnp.int32)
    tile_expert = jnp.concatenate(
        [tile_expert, cum_tiles[-1:].astype(jnp.int32)])

    d1, d2 = pl.pallas_call(
        _dispatch_kernel,
        grid=(n_tok,),
        in_specs=[
            pl.BlockSpec((_TOK_BLK, 1), lambda t: (t, 0)),
            pl.BlockSpec((_TOK_BLK, 1), lambda t: (t, 0)),
            pl.BlockSpec((1, _E), lambda t: (0, 0)),
        ],
        out_specs=[
            pl.BlockSpec((_TOK_BLK, 1), lambda t: (t, 0)),
            pl.BlockSpec((_TOK_BLK, 1), lambda t: (t, 0)),
        ],
        out_shape=[
            jax.ShapeDtypeStruct((_N, 1), jnp.int32),
            jax.ShapeDtypeStruct((_N, 1), jnp.int32),
        ],
        scratch_shapes=[pltpu.VMEM((1, _E), jnp.float32)],
    )(a1, a2, slot_base)

    d1f = d1.reshape(_N)
    d2f = d2.reshape(_N)

    xs = _sc_scatter_call(xf, d1f, d2f)

    n_h = _H // _H_BLK
    os_ = None
    for h in range(n_h):
        w_specs = [
            pl.BlockSpec((1, _H_BLK, _DIM), lambda t, te, h=h: (te[t], h, 0)),
            pl.BlockSpec((1, _H_BLK, _DIM), lambda t, te, h=h: (te[t], h, 0)),
            pl.BlockSpec((1, _DIM, _H_BLK), lambda t, te, h=h: (te[t], 0, h)),
        ]
        xs_spec = pl.BlockSpec((_M_TILE, _DIM), lambda t, te: (t, 0))
        o_spec = pl.BlockSpec((_M_TILE, _DIM), lambda t, te: (t, 0))
        if h == 0:
            os_ = pl.pallas_call(
                _ffn_first_kernel,
                grid_spec=pltpu.PrefetchScalarGridSpec(
                    num_scalar_prefetch=1,
                    grid=(_N_TILES,),
                    in_specs=[xs_spec] + w_specs,
                    out_specs=o_spec,
                ),
                out_shape=jax.ShapeDtypeStruct((_NPAD, _DIM), jnp.float32),
            )(tile_expert, xs, W1, W3, W2)
        else:
            os_ = pl.pallas_call(
                _ffn_acc_kernel,
                grid_spec=pltpu.PrefetchScalarGridSpec(
                    num_scalar_prefetch=1,
                    grid=(_N_TILES,),
                    in_specs=[xs_spec] + w_specs + [o_spec],
                    out_specs=o_spec,
                ),
                out_shape=jax.ShapeDtypeStruct((_NPAD, _DIM), jnp.float32),
                input_output_aliases={5: 0},
            )(tile_expert, xs, W1, W3, W2, os_)

    g1, g2 = _sc_gather_call(os_, d1f, d2f)

    y, losses = pl.pallas_call(
        _combine_kernel,
        grid=(n_tok,),
        in_specs=[
            pl.BlockSpec((_TOK_BLK, _DIM), lambda t: (t, 0)),
            pl.BlockSpec((_TOK_BLK, _DIM), lambda t: (t, 0)),
            pl.BlockSpec((_TOK_BLK, 1), lambda t: (t, 0)),
            pl.BlockSpec((_TOK_BLK, 1), lambda t: (t, 0)),
            pl.BlockSpec((1, 1), lambda t: (0, 0)),
            pl.BlockSpec((1, _E), lambda t: (0, 0)),
        ],
        out_specs=[
            pl.BlockSpec((_TOK_BLK, _DIM), lambda t: (t, 0)),
            pl.BlockSpec((1, 2), lambda t: (0, 0)),
        ],
        out_shape=[
            jax.ShapeDtypeStruct((_N, _DIM), jnp.float32),
            jax.ShapeDtypeStruct((1, 2), jnp.float32),
        ],
    )(g1, g2, p1, p2, zsum, psum)

    return (y.reshape(_B, _S, _DIM), losses[0, 0], losses[0, 1])


# speedup vs baseline: 1.7043x; 1.0089x over previous
import functools

import jax
import jax.numpy as jnp
from jax import lax
from jax.experimental import pallas as pl
from jax.experimental.pallas import tpu as pltpu
from jax.experimental.pallas import tpu_sc as plsc

_B, _S, _DIM = 2, 2048, 1024
_E, _H, _TOPK = 8, 4096, 2
_N = _B * _S
_Z_LOSS_COEF = 0.001

_TOK_BLK = 1024
_H_BLK = 2048
_M_TILE = 256
_N_TILES = (_TOPK * _N) // _M_TILE + _E
_NPAD = _N_TILES * _M_TILE

_SC_NC, _SC_NS = 2, 16
_SC_NW = _SC_NC * _SC_NS
_T_W = _N // _SC_NW
_CH = 64


def _router_kernel(x_ref, wr_ref, a1_ref, a2_ref, p1_ref, p2_ref,
                   cnt_ref, zsum_ref, psum_ref):
    t = pl.program_id(0)
    x = x_ref[...]
    logits = lax.dot_general(
        x, wr_ref[...], (((1,), (1,)), ((), ())),
        preferred_element_type=jnp.float32)

    @pl.when(t == 0)
    def _init():
        cnt_ref[...] = jnp.zeros_like(cnt_ref)
        zsum_ref[...] = jnp.zeros_like(zsum_ref)
        psum_ref[...] = jnp.zeros_like(psum_ref)

    zsum_ref[...] += jnp.sum(logits * logits).reshape(1, 1)

    m1 = jnp.max(logits, axis=-1, keepdims=True)
    ex = jnp.exp(logits - m1)
    probs = ex / jnp.sum(ex, axis=-1, keepdims=True)
    psum_ref[...] += jnp.sum(probs, axis=0, keepdims=True)

    idx = lax.broadcasted_iota(jnp.int32, logits.shape, 1)
    big = jnp.int32(2**30)
    a1 = jnp.min(jnp.where(logits == m1, idx, big), axis=-1, keepdims=True)
    l2 = jnp.where(idx == a1, -jnp.inf, logits)
    m2 = jnp.max(l2, axis=-1, keepdims=True)
    a2 = jnp.min(jnp.where(l2 == m2, idx, big), axis=-1, keepdims=True)
    p1 = jax.nn.sigmoid(m1 - m2)

    a1_ref[...] = a1
    a2_ref[...] = a2
    p1_ref[...] = p1
    p2_ref[...] = 1.0 - p1

    onehot = (jnp.where(idx == a1, 1.0, 0.0)
              + jnp.where(idx == a2, 1.0, 0.0))
    cnt_ref[...] += jnp.sum(onehot, axis=0, keepdims=True).astype(jnp.int32)


def _dispatch_kernel(a1_ref, a2_ref, base_ref, d1_ref, d2_ref, run_ref):
    t = pl.program_id(0)

    @pl.when(t == 0)
    def _init():
        run_ref[...] = jnp.zeros_like(run_ref)

    a1 = a1_ref[...]
    a2 = a2_ref[...]
    lane = lax.broadcasted_iota(jnp.int32, (_TOK_BLK, _E), 1)
    onehot = (jnp.where(lane == a1, 1.0, 0.0)
              + jnp.where(lane == a2, 1.0, 0.0))
    r_io = lax.broadcasted_iota(jnp.int32, (_TOK_BLK, _TOK_BLK), 0)
    c_io = lax.broadcasted_iota(jnp.int32, (_TOK_BLK, _TOK_BLK), 1)
    tril = jnp.where(c_io < r_io, 1.0, 0.0)
    excl = lax.dot_general(
        tril, onehot, (((1,), (0,)), ((), ())),
        preferred_element_type=jnp.float32)
    pos = excl + run_ref[...] + base_ref[...].astype(jnp.float32)
    d1 = jnp.sum(jnp.where(lane == a1, pos, 0.0), axis=1, keepdims=True)
    d2 = jnp.sum(jnp.where(lane == a2, pos, 0.0), axis=1, keepdims=True)
    d1_ref[...] = d1.astype(jnp.int32)
    d2_ref[...] = d2.astype(jnp.int32)
    run_ref[...] += jnp.sum(onehot, axis=0, keepdims=True)


def _sc_scatter_call(xf, d1, d2):
    mesh = plsc.VectorSubcoreMesh(core_axis_name="c", subcore_axis_name="s")

    @functools.partial(
        pl.kernel,
        out_type=jax.ShapeDtypeStruct((_NPAD, _DIM), jnp.float32),
        mesh=mesh,
        scratch_types=[
            pltpu.VMEM((_CH,), jnp.int32),
            pltpu.VMEM((_CH,), jnp.int32),
            pltpu.VMEM((_CH, _DIM), jnp.float32),
        ],
    )
    def scatter_k(x_hbm, d1_hbm, d2_hbm, xs_hbm, idx1_v, idx2_v, rows_v):
        wid = lax.axis_index("s") * _SC_NC + lax.axis_index("c")
        base = wid * _T_W
        for c in range(_T_W // _CH):
            off = base + c * _CH
            pltpu.sync_copy(d1_hbm.at[pl.ds(off, _CH)], idx1_v)
            pltpu.sync_copy(d2_hbm.at[pl.ds(off, _CH)], idx2_v)
            pltpu.sync_copy(x_hbm.at[pl.ds(off, _CH)], rows_v)
            pltpu.sync_copy(rows_v, xs_hbm.at[idx1_v])
            pltpu.sync_copy(rows_v, xs_hbm.at[idx2_v])

    return scatter_k(xf, d1, d2)


def _sc_gather_call(os_, d1, d2):
    mesh = plsc.VectorSubcoreMesh(core_axis_name="c", subcore_axis_name="s")

    @functools.partial(
        pl.kernel,
        out_type=[
            jax.ShapeDtypeStruct((_N, _DIM), jnp.float32),
            jax.ShapeDtypeStruct((_N, _DIM), jnp.float32),
        ],
        mesh=mesh,
        scratch_types=[
            pltpu.VMEM((_CH,), jnp.int32),
            pltpu.VMEM((_CH, _DIM), jnp.float32),
            pltpu.SemaphoreType.DMA,
        ],
    )
    def gather_k(os_hbm, d1_hbm, d2_hbm, g1_hbm, g2_hbm, idx_v, rows_v, sem):
        wid = lax.axis_index("s") * _SC_NC + lax.axis_index("c")
        base = wid * _T_W
        for c in range(_T_W // _CH):
            off = base + c * _CH
            pltpu.sync_copy(d1_hbm.at[pl.ds(off, _CH)], idx_v)
            pltpu.async_copy(os_hbm.at[idx_v], rows_v, sem).wait()
            pltpu.sync_copy(rows_v, g1_hbm.at[pl.ds(off, _CH)])
            pltpu.sync_copy(d2_hbm.at[pl.ds(off, _CH)], idx_v)
            pltpu.async_copy(os_hbm.at[idx_v], rows_v, sem).wait()
            pltpu.sync_copy(rows_v, g2_hbm.at[pl.ds(off, _CH)])

    return gather_k(os_, d1, d2)


def _ffn_part(xs_ref, w1_ref, w3_ref, w2_ref):
    x = xs_ref[...]
    hidden = lax.dot_general(
        x, w1_ref[0], (((1,), (1,)), ((), ())),
        preferred_element_type=jnp.float32)
    gate = lax.dot_general(
        x, w3_ref[0], (((1,), (1,)), ((), ())),
        preferred_element_type=jnp.float32)
    hg = hidden * (gate * jax.nn.sigmoid(gate))
    return lax.dot_general(
        hg, w2_ref[0], (((1,), (1,)), ((), ())),
        preferred_element_type=jnp.float32)


def _ffn_first_kernel(te_ref, xs_ref, w1_ref, w3_ref, w2_ref, os_ref):
    @pl.when(pl.program_id(0) < te_ref[_N_TILES])
    def _compute():
        os_ref[...] = _ffn_part(xs_ref, w1_ref, w3_ref, w2_ref)


def _ffn_acc_kernel(te_ref, xs_ref, w1_ref, w3_ref, w2_ref, oi_ref, os_ref):
    @pl.when(pl.program_id(0) < te_ref[_N_TILES])
    def _compute():
        os_ref[...] = oi_ref[...] + _ffn_part(xs_ref, w1_ref, w3_ref, w2_ref)


def _combine_kernel(g1_ref, g2_ref, p1_ref, p2_ref, zsum_ref, psum_ref,
                    y_ref, loss_ref):
    y_ref[...] = p1_ref[...] * g1_ref[...] + p2_ref[...] * g2_ref[...]

    @pl.when(pl.program_id(0) == 0)
    def _losses():
        z = (zsum_ref[0, 0] / (_N * _E)) * _Z_LOSS_COEF
        pm = psum_ref[...] / _N - 1.0 / _E
        bal = jnp.sum(pm * pm) / _E
        loss_ref[...] = jnp.concatenate(
            [jnp.full((1, 1), z), jnp.full((1, 1), bal)], axis=1)


def kernel(x, Wr, W1, W2, W3):
    xf = x.reshape(_N, _DIM)
    n_tok = _N // _TOK_BLK

    a1, a2, p1, p2, cnt, zsum, psum = pl.pallas_call(
        _router_kernel,
        grid=(n_tok,),
        in_specs=[
            pl.BlockSpec((_TOK_BLK, _DIM), lambda t: (t, 0)),
            pl.BlockSpec((_E, _DIM), lambda t: (0, 0)),
        ],
        out_specs=[
            pl.BlockSpec((_TOK_BLK, 1), lambda t: (t, 0)),
            pl.BlockSpec((_TOK_BLK, 1), lambda t: (t, 0)),
            pl.BlockSpec((_TOK_BLK, 1), lambda t: (t, 0)),
            pl.BlockSpec((_TOK_BLK, 1), lambda t: (t, 0)),
            pl.BlockSpec((1, _E), lambda t: (0, 0)),
            pl.BlockSpec((1, 1), lambda t: (0, 0)),
            pl.BlockSpec((1, _E), lambda t: (0, 0)),
        ],
        out_shape=[
            jax.ShapeDtypeStruct((_N, 1), jnp.int32),
            jax.ShapeDtypeStruct((_N, 1), jnp.int32),
            jax.ShapeDtypeStruct((_N, 1), jnp.float32),
            jax.ShapeDtypeStruct((_N, 1), jnp.float32),
            jax.ShapeDtypeStruct((1, _E), jnp.int32),
            jax.ShapeDtypeStruct((1, 1), jnp.float32),
            jax.ShapeDtypeStruct((1, _E), jnp.float32),
        ],
    )(xf, Wr)

    c = cnt[0]
    tiles = (c + _M_TILE - 1) // _M_TILE
    cum_tiles = jnp.cumsum(tiles)
    slot_base = ((cum_tiles - tiles) * _M_TILE).astype(jnp.int32)[None, :]
    tile_ids = jnp.arange(_N_TILES, dtype=jnp.int32)
    tile_expert = jnp.minimum(
        jnp.sum(tile_ids[:, None] >= cum_tiles[None, :].astype(jnp.int32),
                axis=1),
        _E - 1).astype(jnp.int32)
    tile_expert = jnp.concatenate(
        [tile_expert, cum_tiles[-1:].astype(jnp.int32)])

    d1, d2 = pl.pallas_call(
        _dispatch_kernel,
        grid=(n_tok,),
        in_specs=[
            pl.BlockSpec((_TOK_BLK, 1), lambda t: (t, 0)),
            pl.BlockSpec((_TOK_BLK, 1), lambda t: (t, 0)),
            pl.BlockSpec((1, _E), lambda t: (0, 0)),
        ],
        out_specs=[
            pl.BlockSpec((_TOK_BLK, 1), lambda t: (t, 0)),
            pl.BlockSpec((_TOK_BLK, 1), lambda t: (t, 0)),
        ],
        out_shape=[
            jax.ShapeDtypeStruct((_N, 1), jnp.int32),
            jax.ShapeDtypeStruct((_N, 1), jnp.int32),
        ],
        scratch_shapes=[pltpu.VMEM((1, _E), jnp.float32)],
    )(a1, a2, slot_base)

    d1f = d1.reshape(_N)
    d2f = d2.reshape(_N)

    xs = _sc_scatter_call(xf, d1f, d2f)

    n_h = _H // _H_BLK
    os_ = None
    for h in range(n_h):
        w_specs = [
            pl.BlockSpec((1, _H_BLK, _DIM), lambda t, te, h=h: (te[t], h, 0)),
            pl.BlockSpec((1, _H_BLK, _DIM), lambda t, te, h=h: (te[t], h, 0)),
            pl.BlockSpec((1, _DIM, _H_BLK), lambda t, te, h=h: (te[t], 0, h)),
        ]
        xs_spec = pl.BlockSpec((_M_TILE, _DIM), lambda t, te: (t, 0))
        o_spec = pl.BlockSpec((_M_TILE, _DIM), lambda t, te: (t, 0))
        if h == 0:
            os_ = pl.pallas_call(
                _ffn_first_kernel,
                grid_spec=pltpu.PrefetchScalarGridSpec(
                    num_scalar_prefetch=1,
                    grid=(_N_TILES,),
                    in_specs=[xs_spec] + w_specs,
                    out_specs=o_spec,
                ),
                out_shape=jax.ShapeDtypeStruct((_NPAD, _DIM), jnp.float32),
            )(tile_expert, xs, W1, W3, W2)
        else:
            os_ = pl.pallas_call(
                _ffn_acc_kernel,
                grid_spec=pltpu.PrefetchScalarGridSpec(
                    num_scalar_prefetch=1,
                    grid=(_N_TILES,),
                    in_specs=[xs_spec] + w_specs + [o_spec],
                    out_specs=o_spec,
                ),
                out_shape=jax.ShapeDtypeStruct((_NPAD, _DIM), jnp.float32),
                input_output_aliases={5: 0},
            )(tile_expert, xs, W1, W3, W2, os_)

    g1, g2 = _sc_gather_call(os_, d1f, d2f)

    y, losses = pl.pallas_call(
        _combine_kernel,
        grid=(n_tok,),
        in_specs=[
            pl.BlockSpec((_TOK_BLK, _DIM), lambda t: (t, 0)),
            pl.BlockSpec((_TOK_BLK, _DIM), lambda t: (t, 0)),
            pl.BlockSpec((_TOK_BLK, 1), lambda t: (t, 0)),
            pl.BlockSpec((_TOK_BLK, 1), lambda t: (t, 0)),
            pl.BlockSpec((1, 1), lambda t: (0, 0)),
            pl.BlockSpec((1, _E), lambda t: (0, 0)),
        ],
        out_specs=[
            pl.BlockSpec((_TOK_BLK, _DIM), lambda t: (t, 0)),
            pl.BlockSpec((1, 2), lambda t: (0, 0)),
        ],
        out_shape=[
            jax.ShapeDtypeStruct((_N, _DIM), jnp.float32),
            jax.ShapeDtypeStruct((1, 2), jnp.float32),
        ],
    )(g1, g2, p1, p2, zsum, psum)

    return (y.reshape(_B, _S, _DIM), losses[0, 0], losses[0, 1])
